# baseline (speedup 1.0000x reference)
import functools

import jax
import jax.numpy as jnp
from jax import lax
from jax.experimental import pallas as pl
from jax.experimental.pallas import tpu as pltpu
from jax.experimental.pallas import tpu_sc as plsc

_SC_PARAMS = dict(
    compiler_params=pltpu.CompilerParams(
        needs_layout_passes=False, use_tc_tiling_on_sc=False
    ),
)


def _build_relayout(H, W, D):
    info = plsc.get_sparse_core_info()
    NW = info.num_cores * info.num_subcores
    L = info.num_lanes
    v_per_w = H // NW
    row_words = W * D

    mesh = plsc.VectorSubcoreMesh(core_axis_name="c", subcore_axis_name="s")

    @functools.partial(
        pl.kernel,
        mesh=mesh,
        **_SC_PARAMS,
        out_type=jax.ShapeDtypeStruct((H * W * D,), jnp.float32),
        scratch_types=[
            pltpu.VMEM((2, row_words), jnp.float32),
            pltpu.VMEM((2, row_words), jnp.float32),
            pltpu.SemaphoreType.DMA,
            pltpu.SemaphoreType.DMA,
            pltpu.SemaphoreType.DMA,
            pltpu.SemaphoreType.DMA,
        ],
    )
    def relayout(grid_hbm, table_hbm, in_v, out_v, si0, si1, so0, so1):
        wid = lax.axis_index("s") * info.num_cores + lax.axis_index("c")
        iota = lax.iota(jnp.int32, L)
        sin = (si0, si1)
        sout = (so0, so1)
        vbase = wid * v_per_w

        def fire_in(k, g):
            pltpu.async_copy(grid_hbm.at[vbase + k], in_v.at[g], sin[g])

        def stage(i, k, g):
            pltpu.make_async_copy(grid_hbm.at[vbase + k], in_v.at[g], sin[g]).wait()

            @pl.when(i > 0)
            def _():
                pltpu.make_async_copy(
                    out_v.at[g], table_hbm.at[pl.ds(0, row_words)], sout[g]
                ).wait()

            @plsc.parallel_loop(0, row_words // L, unroll=8)
            def t_body(kk):
                j = kk >> 6
                d = (kk >> 3) & 7
                r0 = (kk & 7) * L
                val = in_v[g, pl.ds(j * 1024 + d * 128 + r0, L)]
                oidx = j * 1024 + r0 * 8 + iota * 8 + d
                plsc.store_scatter(out_v.at[g], [oidx], val)

            pltpu.async_copy(
                out_v.at[g],
                table_hbm.at[pl.ds((vbase + k) * row_words, row_words)],
                sout[g],
            )

            @pl.when(k + 2 < v_per_w)
            def _():
                fire_in(k + 2, g)

        fire_in(0, 0)
        fire_in(1, 1)

        def pipe_body(i, _):
            stage(i, 2 * i, 0)
            stage(i, 2 * i + 1, 1)
            return 0

        lax.fori_loop(0, v_per_w // 2, pipe_body, 0)
        pltpu.make_async_copy(
            out_v.at[0], table_hbm.at[pl.ds(0, row_words)], sout[0]
        ).wait()
        pltpu.make_async_copy(
            out_v.at[1], table_hbm.at[pl.ds(0, row_words)], sout[1]
        ).wait()

    return relayout


def _build_gather(N, H, W, D, C):
    info = plsc.get_sparse_core_info()
    NC, NS, L = info.num_cores, info.num_subcores, info.num_lanes
    NW = NC * NS
    NQP = -(-N // 128) * 128
    span = -(-NQP // NW)
    span += (-span) % 128
    n_chunks = -(-span // C)
    n_chunks += n_chunks % 2
    scale_u = float(W - 1)
    scale_v = float(H - 1)
    row_max = H * W - W - 2

    mesh = plsc.VectorSubcoreMesh(core_axis_name="c", subcore_axis_name="s")

    @functools.partial(
        pl.kernel,
        mesh=mesh,
        **_SC_PARAMS,
        out_type=jax.ShapeDtypeStruct((NQP * D,), jnp.float32),
        scratch_types=[
            pltpu.VMEM((2, C // 128, 2, 128), jnp.float32),
            pltpu.VMEM((2, C), jnp.int32),
            pltpu.VMEM((2, C), jnp.int32),
            pltpu.VMEM((2, C), jnp.int32),
            pltpu.VMEM((2, C), jnp.int32),
            pltpu.VMEM((2, C), jnp.float32),
            pltpu.VMEM((2, C), jnp.float32),
            pltpu.VMEM((2, C, D), jnp.float32),
            pltpu.VMEM((2, C, D), jnp.float32),
            pltpu.VMEM((2, C, D), jnp.float32),
            pltpu.VMEM((2, C, D), jnp.float32),
            pltpu.VMEM((2, C * D), jnp.float32),
            pltpu.SemaphoreType.DMA,
            pltpu.SemaphoreType.DMA,
        ],
    )
    def gather_blend(uv_hbm, table_hbm, out_hbm,
                     uv_v, ia_v, ib_v, ic_v, id_v, al_v, be_v,
                     ra_v, rb_v, rc_v, rd_v, out_v, sem0, sem1):
        wid = lax.axis_index("s") * NC + lax.axis_index("c")
        iota = lax.iota(jnp.int32, L)
        sems = (sem0, sem1)

        wbase = jnp.minimum(wid * span, NQP - span)

        def prep(ci, g):
            base = wbase + jnp.minimum(ci * C, span - C)
            pltpu.sync_copy(uv_hbm.at[pl.ds(base // 128, C // 128)], uv_v.at[g])

            @plsc.parallel_loop(0, C // L, unroll=8)
            def idx_body(t):
                off = t * L
                u = uv_v[g, off >> 7, 0, pl.ds(off & 127, L)]
                v = uv_v[g, off >> 7, 1, pl.ds(off & 127, L)]
                xf = u * scale_u
                ix = xf.astype(jnp.int32)
                fx = xf - ix.astype(jnp.float32)
                yf = v * scale_v
                iy = yf.astype(jnp.int32)
                fy = yf - iy.astype(jnp.float32)
                row = jnp.minimum(jnp.maximum(iy * W + ix, 0), row_max)
                ia_v[g, pl.ds(off, L)] = row
                ib_v[g, pl.ds(off, L)] = row + 1
                ic_v[g, pl.ds(off, L)] = row + W
                id_v[g, pl.ds(off, L)] = row + (W + 1)
                al_v[g, pl.ds(off, L)] = fx
                be_v[g, pl.ds(off, L)] = fy

            pltpu.async_copy(table_hbm.at[ia_v.at[g]], ra_v.at[g], sems[g])
            pltpu.async_copy(table_hbm.at[ib_v.at[g]], rb_v.at[g], sems[g])
            pltpu.async_copy(table_hbm.at[ic_v.at[g]], rc_v.at[g], sems[g])
            pltpu.async_copy(table_hbm.at[id_v.at[g]], rd_v.at[g], sems[g])

        def blend(ci, g):
            base = wbase + jnp.minimum(ci * C, span - C)
            pltpu.make_async_copy(table_hbm.at[ia_v.at[g]], ra_v.at[g], sems[g]).wait()
            pltpu.make_async_copy(table_hbm.at[ib_v.at[g]], rb_v.at[g], sems[g]).wait()
            pltpu.make_async_copy(table_hbm.at[ic_v.at[g]], rc_v.at[g], sems[g]).wait()
            pltpu.make_async_copy(table_hbm.at[id_v.at[g]], rd_v.at[g], sems[g]).wait()

            @plsc.parallel_loop(0, C // L, unroll=4)
            def blend_body(k):
                b = k >> 3
                s = k & 7
                qs = b * 128 + s * L
                rid = qs + iota
                al = al_v[g, pl.ds(qs, L)]
                be = be_v[g, pl.ds(qs, L)]
                nal = 1.0 - al
                nbe = 1.0 - be
                wa = nal * nbe
                wb = al * nbe
                wc = nal * be
                wd = al * be
                for d in range(D):
                    dvec = iota * 0 + d
                    a = plsc.load_gather(ra_v.at[g], [rid, dvec])
                    bb = plsc.load_gather(rb_v.at[g], [rid, dvec])
                    c = plsc.load_gather(rc_v.at[g], [rid, dvec])
                    dd = plsc.load_gather(rd_v.at[g], [rid, dvec])
                    o = a * wa + bb * wb + c * wc + dd * wd
                    out_v[g, pl.ds(b * 1024 + d * 128 + s * L, L)] = o

            pltpu.sync_copy(out_v.at[g], out_hbm.at[pl.ds(base * D, C * D)])

        prep(0, 0)

        def pipe_body(i, _):
            prep(2 * i + 1, 1)
            blend(2 * i, 0)

            @pl.when(i < n_chunks // 2 - 1)
            def _():
                prep(2 * i + 2, 0)

            blend(2 * i + 1, 1)
            return 0

        lax.fori_loop(0, n_chunks // 2, pipe_body, 0)

    return gather_blend


def kernel(uvList, grid):
    N = uvList.shape[0]
    H, W, D = grid.shape
    grid_native = (
        grid.transpose(0, 2, 1)
        .reshape(H, D, W // 128, 128)
        .transpose(0, 2, 1, 3)
        .reshape(H, W * D)
    )
    table_flat = _build_relayout(H, W, D)(grid_native)
    table = table_flat.reshape(H * W, D)
    NQP = -(-N // 128) * 128
    uv_native = (
        jnp.pad(uvList, ((0, NQP - N), (0, 0)))
        .reshape(NQP // 128, 128, 2)
        .transpose(0, 2, 1)
    )
    out_flat = _build_gather(N, H, W, D, 1024)(uv_native, table)
    NQP = -(-N // 128) * 128
    out = (
        out_flat.reshape(NQP // 128, D, 128)
        .transpose(0, 2, 1)
        .reshape(NQP, D)
    )
    return out[:N]


# speedup vs baseline: 1.0424x; 1.0424x over previous
import functools

import jax
import jax.numpy as jnp
from jax import lax
from jax.experimental import pallas as pl
from jax.experimental.pallas import tpu as pltpu
from jax.experimental.pallas import tpu_sc as plsc

_SC_PARAMS = dict(
    compiler_params=pltpu.CompilerParams(
        needs_layout_passes=False, use_tc_tiling_on_sc=False
    ),
)


def _build_relayout(H, W, D):
    info = plsc.get_sparse_core_info()
    NW = info.num_cores * info.num_subcores
    L = info.num_lanes
    v_per_w = H // NW
    row_words = W * D

    mesh = plsc.VectorSubcoreMesh(core_axis_name="c", subcore_axis_name="s")

    @functools.partial(
        pl.kernel,
        mesh=mesh,
        **_SC_PARAMS,
        out_type=jax.ShapeDtypeStruct((H * W * D,), jnp.float32),
        scratch_types=[
            pltpu.VMEM((2, row_words), jnp.float32),
            pltpu.VMEM((2, row_words), jnp.float32),
            pltpu.SemaphoreType.DMA,
            pltpu.SemaphoreType.DMA,
            pltpu.SemaphoreType.DMA,
            pltpu.SemaphoreType.DMA,
        ],
    )
    def relayout(grid_hbm, table_hbm, in_v, out_v, si0, si1, so0, so1):
        wid = lax.axis_index("s") * info.num_cores + lax.axis_index("c")
        iota = lax.iota(jnp.int32, L)
        sin = (si0, si1)
        sout = (so0, so1)
        vbase = wid * v_per_w

        def fire_in(k, g):
            pltpu.async_copy(grid_hbm.at[vbase + k], in_v.at[g], sin[g])

        def stage(i, k, g):
            pltpu.make_async_copy(grid_hbm.at[vbase + k], in_v.at[g], sin[g]).wait()

            @pl.when(i > 0)
            def _():
                pltpu.make_async_copy(
                    out_v.at[g], table_hbm.at[pl.ds(0, row_words)], sout[g]
                ).wait()

            @plsc.parallel_loop(0, row_words // L, unroll=8)
            def t_body(kk):
                j = kk >> 6
                d = (kk >> 3) & 7
                r0 = (kk & 7) * L
                val = in_v[g, pl.ds(j * 1024 + d * 128 + r0, L)]
                oidx = j * 1024 + r0 * 8 + iota * 8 + d
                plsc.store_scatter(out_v.at[g], [oidx], val)

            pltpu.async_copy(
                out_v.at[g],
                table_hbm.at[pl.ds((vbase + k) * row_words, row_words)],
                sout[g],
            )

            @pl.when(k + 2 < v_per_w)
            def _():
                fire_in(k + 2, g)

        fire_in(0, 0)
        fire_in(1, 1)

        def pipe_body(i, _):
            stage(i, 2 * i, 0)
            stage(i, 2 * i + 1, 1)
            return 0

        lax.fori_loop(0, v_per_w // 2, pipe_body, 0)
        pltpu.make_async_copy(
            out_v.at[0], table_hbm.at[pl.ds(0, row_words)], sout[0]
        ).wait()
        pltpu.make_async_copy(
            out_v.at[1], table_hbm.at[pl.ds(0, row_words)], sout[1]
        ).wait()

    return relayout


def _build_gather(N, H, W, D, C):
    info = plsc.get_sparse_core_info()
    NC, NS, L = info.num_cores, info.num_subcores, info.num_lanes
    NW = NC * NS
    NQP = -(-N // 128) * 128
    span = -(-NQP // NW)
    span += (-span) % 128
    n_chunks = -(-span // C)
    n_chunks += n_chunks % 2
    scale_u = float(W - 1)
    scale_v = float(H - 1)
    row_max = H * W - W - 2

    mesh = plsc.VectorSubcoreMesh(core_axis_name="c", subcore_axis_name="s")

    @functools.partial(
        pl.kernel,
        mesh=mesh,
        **_SC_PARAMS,
        out_type=jax.ShapeDtypeStruct((NQP * D,), jnp.float32),
        scratch_types=[
            pltpu.VMEM((2, C // 128, 2, 128), jnp.float32),
            pltpu.VMEM((2, C), jnp.int32),
            pltpu.VMEM((2, C), jnp.int32),
            pltpu.VMEM((2, C), jnp.int32),
            pltpu.VMEM((2, C), jnp.int32),
            pltpu.VMEM((2, C), jnp.float32),
            pltpu.VMEM((2, C), jnp.float32),
            pltpu.VMEM((2, C, D), jnp.float32),
            pltpu.VMEM((2, C, D), jnp.float32),
            pltpu.VMEM((2, C, D), jnp.float32),
            pltpu.VMEM((2, C, D), jnp.float32),
            pltpu.VMEM((2, C * D), jnp.float32),
            pltpu.SemaphoreType.DMA,
            pltpu.SemaphoreType.DMA,
        ],
    )
    def gather_blend(uv_hbm, table_hbm, out_hbm,
                     uv_v, ia_v, ib_v, ic_v, id_v, al_v, be_v,
                     ra_v, rb_v, rc_v, rd_v, out_v, sem0, sem1):
        wid = lax.axis_index("s") * NC + lax.axis_index("c")
        iota = lax.iota(jnp.int32, L)
        sems = (sem0, sem1)

        wbase = jnp.minimum(wid * span, NQP - span)

        def prep(ci, g):
            base = wbase + jnp.minimum(ci * C, span - C)
            pltpu.sync_copy(uv_hbm.at[pl.ds(base // 128, C // 128)], uv_v.at[g])

            @plsc.parallel_loop(0, C // L, unroll=4)
            def idx_body(t):
                off = t * L
                u = uv_v[g, off >> 7, 0, pl.ds(off & 127, L)]
                v = uv_v[g, off >> 7, 1, pl.ds(off & 127, L)]
                xf = u * scale_u
                ix = xf.astype(jnp.int32)
                fx = xf - ix.astype(jnp.float32)
                yf = v * scale_v
                iy = yf.astype(jnp.int32)
                fy = yf - iy.astype(jnp.float32)
                row = jnp.minimum(jnp.maximum(iy * W + ix, 0), row_max)
                ia_v[g, pl.ds(off, L)] = row
                ib_v[g, pl.ds(off, L)] = row + 1
                ic_v[g, pl.ds(off, L)] = row + W
                id_v[g, pl.ds(off, L)] = row + (W + 1)
                al_v[g, pl.ds(off, L)] = fx
                be_v[g, pl.ds(off, L)] = fy

            pltpu.async_copy(table_hbm.at[ia_v.at[g]], ra_v.at[g], sems[g])
            pltpu.async_copy(table_hbm.at[ib_v.at[g]], rb_v.at[g], sems[g])
            pltpu.async_copy(table_hbm.at[ic_v.at[g]], rc_v.at[g], sems[g])
            pltpu.async_copy(table_hbm.at[id_v.at[g]], rd_v.at[g], sems[g])

        def blend(ci, g):
            base = wbase + jnp.minimum(ci * C, span - C)
            pltpu.make_async_copy(table_hbm.at[ia_v.at[g]], ra_v.at[g], sems[g]).wait()
            pltpu.make_async_copy(table_hbm.at[ib_v.at[g]], rb_v.at[g], sems[g]).wait()
            pltpu.make_async_copy(table_hbm.at[ic_v.at[g]], rc_v.at[g], sems[g]).wait()
            pltpu.make_async_copy(table_hbm.at[id_v.at[g]], rd_v.at[g], sems[g]).wait()

            @plsc.parallel_loop(0, C // L, unroll=2)
            def blend_body(k):
                b = k >> 3
                s = k & 7
                qs = b * 128 + s * L
                rid = qs + iota
                al = al_v[g, pl.ds(qs, L)]
                be = be_v[g, pl.ds(qs, L)]
                nal = 1.0 - al
                nbe = 1.0 - be
                wa = nal * nbe
                wb = al * nbe
                wc = nal * be
                wd = al * be
                for d in range(D):
                    dvec = iota * 0 + d
                    a = plsc.load_gather(ra_v.at[g], [rid, dvec])
                    bb = plsc.load_gather(rb_v.at[g], [rid, dvec])
                    c = plsc.load_gather(rc_v.at[g], [rid, dvec])
                    dd = plsc.load_gather(rd_v.at[g], [rid, dvec])
                    o = a * wa + bb * wb + c * wc + dd * wd
                    out_v[g, pl.ds(b * 1024 + d * 128 + s * L, L)] = o

            pltpu.sync_copy(out_v.at[g], out_hbm.at[pl.ds(base * D, C * D)])

        prep(0, 0)

        def pipe_body(i, _):
            prep(2 * i + 1, 1)
            blend(2 * i, 0)

            @pl.when(i < n_chunks // 2 - 1)
            def _():
                prep(2 * i + 2, 0)

            blend(2 * i + 1, 1)
            return 0

        lax.fori_loop(0, n_chunks // 2, pipe_body, 0)

    return gather_blend


def kernel(uvList, grid):
    N = uvList.shape[0]
    H, W, D = grid.shape
    grid_native = (
        grid.transpose(0, 2, 1)
        .reshape(H, D, W // 128, 128)
        .transpose(0, 2, 1, 3)
        .reshape(H, W * D)
    )
    table_flat = _build_relayout(H, W, D)(grid_native)
    table = table_flat.reshape(H * W, D)
    NQP = -(-N // 128) * 128
    uv_native = (
        jnp.pad(uvList, ((0, NQP - N), (0, 0)))
        .reshape(NQP // 128, 128, 2)
        .transpose(0, 2, 1)
    )
    out_flat = _build_gather(N, H, W, D, 1024)(uv_native, table)
    NQP = -(-N // 128) * 128
    out = (
        out_flat.reshape(NQP // 128, D, 128)
        .transpose(0, 2, 1)
        .reshape(NQP, D)
    )
    return out[:N]
